# baseline (speedup 1.0000x reference)
import functools

import jax
import jax.numpy as jnp
from jax import lax
from jax.experimental import pallas as pl
from jax.experimental.pallas import tpu as pltpu
from jax.experimental.pallas import tpu_sc as plsc

N = 10000
E = 160000
NC = 2
NS = 16
NW = NC * NS

_SC_PARAMS = pltpu.CompilerParams(needs_layout_passes=False)


def _dense_node_kernel(f_ref, pos_ref, waxT_ref, wapT_ref, ba_ref, wbT_ref,
                       bb_ref, u_ref, v_ref, s_ref):
    a = jnp.dot(f_ref[...], waxT_ref[...],
                preferred_element_type=jnp.float32) + ba_ref[...]
    p = jnp.dot(pos_ref[...], wapT_ref[...],
                preferred_element_type=jnp.float32)
    u_ref[...] = a + p
    v_ref[...] = p
    s_ref[...] = jnp.dot(jax.nn.relu(a), wbT_ref[...],
                         preferred_element_type=jnp.float32) + bb_ref[...]


def _edge_mlp_kernel(us_ref, vd_ref, wbT_ref, bb_ref, m_ref):
    t = jax.nn.relu(us_ref[...] - vd_ref[...])
    m_ref[...] = jnp.dot(t, wbT_ref[...],
                         preferred_element_type=jnp.float32) + bb_ref[...]


def _dense_node(f, pos, waxT, wapT, ba, wbT, bb):
    n, din = f.shape
    hu = waxT.shape[1]
    hs = wbT.shape[1]
    bn = 1000
    grid = (n // bn,)
    uv_sds = jax.ShapeDtypeStruct((n, hu), jnp.float32)
    s_sds = jax.ShapeDtypeStruct((n, hs), jnp.float32)
    u, v, s = pl.pallas_call(
        _dense_node_kernel,
        grid=grid,
        in_specs=[
            pl.BlockSpec((bn, din), lambda i: (i, 0)),
            pl.BlockSpec((bn, 3), lambda i: (i, 0)),
            pl.BlockSpec((din, hu), lambda i: (0, 0)),
            pl.BlockSpec((3, hu), lambda i: (0, 0)),
            pl.BlockSpec((1, hu), lambda i: (0, 0)),
            pl.BlockSpec((hu, hs), lambda i: (0, 0)),
            pl.BlockSpec((1, hs), lambda i: (0, 0)),
        ],
        out_specs=[
            pl.BlockSpec((bn, hu), lambda i: (i, 0)),
            pl.BlockSpec((bn, hu), lambda i: (i, 0)),
            pl.BlockSpec((bn, hs), lambda i: (i, 0)),
        ],
        out_shape=[uv_sds, uv_sds, s_sds],
    )(f, pos, waxT, wapT, ba, wbT, bb)
    return u, v, s


def _edge_mlp(us, vd, wbT, bb):
    e, hu = us.shape
    hs = wbT.shape[1]
    be = 2000
    grid = (e // be,)
    m = pl.pallas_call(
        _edge_mlp_kernel,
        grid=grid,
        in_specs=[
            pl.BlockSpec((be, hu), lambda i: (i, 0)),
            pl.BlockSpec((be, hu), lambda i: (i, 0)),
            pl.BlockSpec((hu, hs), lambda i: (0, 0)),
            pl.BlockSpec((1, hs), lambda i: (0, 0)),
        ],
        out_specs=pl.BlockSpec((be, hs), lambda i: (i, 0)),
        out_shape=jax.ShapeDtypeStruct((e, hs), jnp.float32),
    )(us, vd, wbT, bb)
    return m


def _edge_gather(u, v, src, dst):
    h = u.shape[1]
    per_w = E // NW
    cg = 200
    n_chunks = per_w // cg
    mesh = plsc.VectorSubcoreMesh(core_axis_name="c", subcore_axis_name="s")
    sds = jax.ShapeDtypeStruct((E, h), jnp.float32)

    @functools.partial(
        pl.kernel, mesh=mesh,
        out_type=[sds, sds],
        compiler_params=_SC_PARAMS,
        scratch_types=[
            pltpu.VMEM((cg,), jnp.int32),
            pltpu.VMEM((cg,), jnp.int32),
            pltpu.VMEM((cg, h), jnp.float32),
            pltpu.VMEM((cg, h), jnp.float32),
            pltpu.SemaphoreType.DMA,
            pltpu.SemaphoreType.DMA,
        ],
    )
    def k(u_hbm, v_hbm, src_hbm, dst_hbm, us_hbm, vd_hbm,
          sidx, didx, ubuf, vbuf, sem1, sem2):
        wid = lax.axis_index("s") * NC + lax.axis_index("c")
        base = wid * per_w

        def body(c, _):
            off = base + c * cg
            pltpu.sync_copy(src_hbm.at[pl.ds(off, cg)], sidx)
            pltpu.sync_copy(dst_hbm.at[pl.ds(off, cg)], didx)
            cp1 = pltpu.async_copy(u_hbm.at[sidx], ubuf, sem1)
            cp2 = pltpu.async_copy(v_hbm.at[didx], vbuf, sem2)
            cp1.wait()
            cp2.wait()
            pltpu.sync_copy(ubuf, us_hbm.at[pl.ds(off, cg)])
            pltpu.sync_copy(vbuf, vd_hbm.at[pl.ds(off, cg)])
            return ()

        lax.fori_loop(0, n_chunks, body, (), unroll=False)

    return k(u, v, src, dst)


R = 320
NP = NW * R
KE = 32
CH = 2000
CAP = E + 4096


def _partition(dst):
    mesh = plsc.VectorSubcoreMesh(core_axis_name="c", subcore_axis_name="s")
    n_chunks = E // CH
    n_vecs = CH // 16

    @functools.partial(
        pl.kernel, mesh=mesh,
        out_type=[
            jax.ShapeDtypeStruct((NW * CAP,), jnp.int32),
            jax.ShapeDtypeStruct((NW * CAP,), jnp.int32),
            jax.ShapeDtypeStruct((NW * 16,), jnp.int32),
        ],
        compiler_params=_SC_PARAMS,
        scratch_types=[
            pltpu.VMEM((CH,), jnp.int32),
            pltpu.VMEM((CH + 16,), jnp.int32),
            pltpu.VMEM((CH + 16,), jnp.int32),
            pltpu.VMEM((16,), jnp.int32),
        ],
    )
    def k(dst_hbm, eids_hbm, ldst_hbm, cnt_hbm, dbuf, ebuf, lbuf, cbuf):
        wid = lax.axis_index("s") * NC + lax.axis_index("c")
        lo = wid * R
        lane = lax.iota(jnp.int32, 16)
        ones = jnp.ones((16,), jnp.int32)

        def chunk(c, goff):
            pltpu.sync_copy(dst_hbm.at[pl.ds(c * CH, CH)], dbuf)

            def vec(i, lcnt):
                d = dbuf[pl.ds(i * 16, 16)]
                mask = (d >= lo) & (d < lo + R)
                eid = c * CH + i * 16 + lane
                pos = plsc.cumsum(jnp.where(mask, ones, 0))
                idx = lcnt + pos - 1
                plsc.store_scatter(ebuf, [idx], eid, mask=mask)
                plsc.store_scatter(lbuf, [idx], d - lo, mask=mask)
                return lcnt + plsc.all_reduce_population_count(mask)

            lcnt = lax.fori_loop(0, n_vecs, vec, jnp.zeros((16,), jnp.int32),
                                 unroll=False)
            lc = lcnt[0]
            ebuf[pl.ds(lc, 16)] = jnp.zeros((16,), jnp.int32)
            lbuf[pl.ds(lc, 16)] = jnp.full((16,), R, jnp.int32)
            lc_pad = ((lc + 7) // 8) * 8
            go8 = pl.multiple_of(wid * CAP + goff, 8)
            pltpu.sync_copy(ebuf.at[pl.ds(0, CH)],
                            eids_hbm.at[pl.ds(go8, CH)])
            pltpu.sync_copy(lbuf.at[pl.ds(0, CH)],
                            ldst_hbm.at[pl.ds(go8, CH)])
            return goff + lc_pad

        cnt = lax.fori_loop(0, n_chunks, chunk, jnp.int32(0), unroll=False)
        for t in range(KE // 16):
            ebuf[pl.ds(t * 16, 16)] = jnp.zeros((16,), jnp.int32)
            lbuf[pl.ds(t * 16, 16)] = jnp.full((16,), R, jnp.int32)
        cnt8 = pl.multiple_of(wid * CAP + cnt, 8)
        pltpu.sync_copy(ebuf.at[pl.ds(0, KE)],
                        eids_hbm.at[pl.ds(cnt8, KE)])
        pltpu.sync_copy(lbuf.at[pl.ds(0, KE)],
                        ldst_hbm.at[pl.ds(cnt8, KE)])
        cbuf[...] = jnp.broadcast_to(cnt, (16,))
        pltpu.sync_copy(cbuf, cnt_hbm.at[pl.ds(wid * 16, 16)])

    return k(dst)


def _scatter_max(m, s_pad, eids, ldst, cnts):
    h = m.shape[1]
    mesh = plsc.VectorSubcoreMesh(core_axis_name="c", subcore_axis_name="s")

    @functools.partial(
        pl.kernel, mesh=mesh,
        out_type=jax.ShapeDtypeStruct((NP, h), jnp.float32),
        compiler_params=_SC_PARAMS,
        scratch_types=[
            pltpu.VMEM((R + 1, h), jnp.float32),
            pltpu.VMEM((KE, h), jnp.float32),
            pltpu.VMEM((KE,), jnp.int32),
            pltpu.VMEM((KE,), jnp.int32),
            pltpu.VMEM((16,), jnp.int32),
            pltpu.SemaphoreType.DMA,
        ],
    )
    def k(m_hbm, s_hbm, eids_hbm, ldst_hbm, cnt_hbm, out_hbm,
          acc, mrows, eibuf, ldbuf, cbuf, sem):
        wid = lax.axis_index("s") * NC + lax.axis_index("c")
        lo = wid * R
        pltpu.sync_copy(s_hbm.at[pl.ds(lo, R)], acc.at[pl.ds(0, R)])
        pltpu.sync_copy(cnt_hbm.at[pl.ds(wid * 16, 16)], cbuf)
        cnt = cbuf[pl.ds(0, 16)][0]
        nblk = (cnt + KE - 1) // KE

        def chunk(ck, _):
            off = wid * CAP + ck * KE
            pltpu.sync_copy(eids_hbm.at[pl.ds(off, KE)], eibuf)
            pltpu.sync_copy(ldst_hbm.at[pl.ds(off, KE)], ldbuf)
            pltpu.async_copy(m_hbm.at[eibuf], mrows, sem).wait()
            for b in range(KE // 16):
                ldv = ldbuf[pl.ds(b * 16, 16)]
                for jj in range(16):
                    d = ldv[jj]
                    for hk in range(h // 16):
                        sl = pl.ds(hk * 16, 16)
                        acc[d, sl] = jnp.maximum(acc[d, sl],
                                                 mrows[b * 16 + jj, sl])
            return ()

        lax.fori_loop(0, nblk, chunk, (), unroll=False)
        pltpu.sync_copy(acc.at[pl.ds(0, R)], out_hbm.at[pl.ds(lo, R)])

    return k(m, s_pad, eids, ldst, cnts)


def _layer(f, pos, src, dst, part, Wa, ba, Wb, bb):
    din = f.shape[1]
    waxT = Wa[:, :din].T
    wapT = Wa[:, din:].T
    wbT = Wb.T
    hh = Wa.shape[0]
    hp = max(hh, 128)
    if hp != hh:
        pad = ((0, 0), (0, hp - hh))
        waxT = jnp.pad(waxT, pad)
        wapT = jnp.pad(wapT, pad)
        ba = jnp.pad(ba, (0, hp - hh))
        wbT = jnp.pad(wbT, ((0, hp - hh), (0, hp - hh)))
        bb = jnp.pad(bb, (0, hp - hh))
    u, v, s = _dense_node(f, pos, waxT, wapT, ba.reshape(1, -1), wbT,
                          bb.reshape(1, -1))
    us, vd = _edge_gather(u, v, src, dst)
    m = _edge_mlp(us, vd, wbT, bb.reshape(1, -1))
    s_pad = jnp.pad(s, ((0, NP - s.shape[0]), (0, 0)))
    h_pad = _scatter_max(m, s_pad, *part)
    return h_pad[:N, :hh]


def kernel(x, pos, edge_index, W1a, b1a, W1b, b1b, W2a, b2a, W2b, b2b,
           W3a, b3a, W3b, b3b):
    src = edge_index[0]
    dst = edge_index[1]
    part = _partition(dst)
    h = _layer(x, pos, src, dst, part, W1a, b1a, W1b, b1b)
    h = _layer(h, pos, src, dst, part, W2a, b2a, W2b, b2b)
    h = _layer(h, pos, src, dst, part, W3a, b3a, W3b, b3b)
    return h


# speedup vs baseline: 1.1931x; 1.1931x over previous
import functools

import jax
import jax.numpy as jnp
from jax import lax
from jax.experimental import pallas as pl
from jax.experimental.pallas import tpu as pltpu
from jax.experimental.pallas import tpu_sc as plsc

N = 10000
E = 160000
NC = 2
NS = 16
NW = NC * NS

_SC_PARAMS = pltpu.CompilerParams(needs_layout_passes=False)


def _dense_node_kernel(f_ref, pos_ref, waxT_ref, wapT_ref, ba_ref, wbT_ref,
                       bb_ref, u_ref, v_ref, s_ref):
    a = jnp.dot(f_ref[...], waxT_ref[...],
                preferred_element_type=jnp.float32) + ba_ref[...]
    p = jnp.dot(pos_ref[...], wapT_ref[...],
                preferred_element_type=jnp.float32)
    u_ref[...] = a + p
    v_ref[...] = p
    s_ref[...] = jnp.dot(jax.nn.relu(a), wbT_ref[...],
                         preferred_element_type=jnp.float32) + bb_ref[...]


def _edge_mlp_kernel(us_ref, vd_ref, wbT_ref, bb_ref, m_ref):
    t = jax.nn.relu(us_ref[...] - vd_ref[...])
    m_ref[...] = jnp.dot(t, wbT_ref[...],
                         preferred_element_type=jnp.float32) + bb_ref[...]


def _dense_node(f, pos, waxT, wapT, ba, wbT, bb):
    n, din = f.shape
    hu = waxT.shape[1]
    hs = wbT.shape[1]
    bn = 1000
    grid = (n // bn,)
    uv_sds = jax.ShapeDtypeStruct((n, hu), jnp.float32)
    s_sds = jax.ShapeDtypeStruct((n, hs), jnp.float32)
    u, v, s = pl.pallas_call(
        _dense_node_kernel,
        grid=grid,
        in_specs=[
            pl.BlockSpec((bn, din), lambda i: (i, 0)),
            pl.BlockSpec((bn, 3), lambda i: (i, 0)),
            pl.BlockSpec((din, hu), lambda i: (0, 0)),
            pl.BlockSpec((3, hu), lambda i: (0, 0)),
            pl.BlockSpec((1, hu), lambda i: (0, 0)),
            pl.BlockSpec((hu, hs), lambda i: (0, 0)),
            pl.BlockSpec((1, hs), lambda i: (0, 0)),
        ],
        out_specs=[
            pl.BlockSpec((bn, hu), lambda i: (i, 0)),
            pl.BlockSpec((bn, hu), lambda i: (i, 0)),
            pl.BlockSpec((bn, hs), lambda i: (i, 0)),
        ],
        out_shape=[uv_sds, uv_sds, s_sds],
    )(f, pos, waxT, wapT, ba, wbT, bb)
    return u, v, s


def _edge_mlp(us, vd, wbT, bb):
    e, hu = us.shape
    hs = wbT.shape[1]
    be = 2000
    grid = (e // be,)
    m = pl.pallas_call(
        _edge_mlp_kernel,
        grid=grid,
        in_specs=[
            pl.BlockSpec((be, hu), lambda i: (i, 0)),
            pl.BlockSpec((be, hu), lambda i: (i, 0)),
            pl.BlockSpec((hu, hs), lambda i: (0, 0)),
            pl.BlockSpec((1, hs), lambda i: (0, 0)),
        ],
        out_specs=pl.BlockSpec((be, hs), lambda i: (i, 0)),
        out_shape=jax.ShapeDtypeStruct((e, hs), jnp.float32),
    )(us, vd, wbT, bb)
    return m


def _edge_gather(u, v, src, dst):
    h = u.shape[1]
    per_w = E // NW
    cg = 200
    n_chunks = per_w // cg
    mesh = plsc.VectorSubcoreMesh(core_axis_name="c", subcore_axis_name="s")
    sds = jax.ShapeDtypeStruct((E, h), jnp.float32)

    @functools.partial(
        pl.kernel, mesh=mesh,
        out_type=[sds, sds],
        compiler_params=_SC_PARAMS,
        scratch_types=[
            pltpu.VMEM((cg,), jnp.int32),
            pltpu.VMEM((cg,), jnp.int32),
            pltpu.VMEM((cg, h), jnp.float32),
            pltpu.VMEM((cg, h), jnp.float32),
            pltpu.SemaphoreType.DMA,
            pltpu.SemaphoreType.DMA,
        ],
    )
    def k(u_hbm, v_hbm, src_hbm, dst_hbm, us_hbm, vd_hbm,
          sidx, didx, ubuf, vbuf, sem1, sem2):
        wid = lax.axis_index("s") * NC + lax.axis_index("c")
        base = wid * per_w

        def body(c, _):
            off = base + c * cg
            pltpu.sync_copy(src_hbm.at[pl.ds(off, cg)], sidx)
            pltpu.sync_copy(dst_hbm.at[pl.ds(off, cg)], didx)
            cp1 = pltpu.async_copy(u_hbm.at[sidx], ubuf, sem1)
            cp2 = pltpu.async_copy(v_hbm.at[didx], vbuf, sem2)
            cp1.wait()
            cp2.wait()
            pltpu.sync_copy(ubuf, us_hbm.at[pl.ds(off, cg)])
            pltpu.sync_copy(vbuf, vd_hbm.at[pl.ds(off, cg)])
            return ()

        lax.fori_loop(0, n_chunks, body, (), unroll=False)

    return k(u, v, src, dst)


R = 320
NP = NW * R
KE = 128
CH = 8000
CAP = E + 4096


def _partition(dst):
    mesh = plsc.VectorSubcoreMesh(core_axis_name="c", subcore_axis_name="s")
    n_chunks = E // CH
    n_vecs = CH // 16

    @functools.partial(
        pl.kernel, mesh=mesh,
        out_type=[
            jax.ShapeDtypeStruct((NW * CAP,), jnp.int32),
            jax.ShapeDtypeStruct((NW * CAP,), jnp.int32),
            jax.ShapeDtypeStruct((NW * 16,), jnp.int32),
        ],
        compiler_params=_SC_PARAMS,
        scratch_types=[
            pltpu.VMEM((CH,), jnp.int32),
            pltpu.VMEM((CH + 16,), jnp.int32),
            pltpu.VMEM((CH + 16,), jnp.int32),
            pltpu.VMEM((16,), jnp.int32),
        ],
    )
    def k(dst_hbm, eids_hbm, ldst_hbm, cnt_hbm, dbuf, ebuf, lbuf, cbuf):
        wid = lax.axis_index("s") * NC + lax.axis_index("c")
        lo = wid * R
        lane = lax.iota(jnp.int32, 16)
        ones = jnp.ones((16,), jnp.int32)

        def chunk(c, goff):
            pltpu.sync_copy(dst_hbm.at[pl.ds(c * CH, CH)], dbuf)

            def vec(i, lcnt):
                d = dbuf[pl.ds(i * 16, 16)]
                mask = (d >= lo) & (d < lo + R)
                eid = c * CH + i * 16 + lane
                pos = plsc.cumsum(jnp.where(mask, ones, 0))
                idx = lcnt + pos - 1
                plsc.store_scatter(ebuf, [idx], eid, mask=mask)
                plsc.store_scatter(lbuf, [idx], d - lo, mask=mask)
                return lcnt + plsc.all_reduce_population_count(mask)

            lcnt = lax.fori_loop(0, n_vecs, vec, jnp.zeros((16,), jnp.int32),
                                 unroll=False)
            lc = lcnt[0]
            ebuf[pl.ds(lc, 16)] = jnp.zeros((16,), jnp.int32)
            lbuf[pl.ds(lc, 16)] = jnp.full((16,), R, jnp.int32)
            lc_pad = ((lc + 7) // 8) * 8
            go8 = pl.multiple_of(wid * CAP + goff, 8)
            pltpu.sync_copy(ebuf.at[pl.ds(0, CH)],
                            eids_hbm.at[pl.ds(go8, CH)])
            pltpu.sync_copy(lbuf.at[pl.ds(0, CH)],
                            ldst_hbm.at[pl.ds(go8, CH)])
            return goff + lc_pad

        cnt = lax.fori_loop(0, n_chunks, chunk, jnp.int32(0), unroll=False)
        for t in range(KE // 16):
            ebuf[pl.ds(t * 16, 16)] = jnp.zeros((16,), jnp.int32)
            lbuf[pl.ds(t * 16, 16)] = jnp.full((16,), R, jnp.int32)
        cnt8 = pl.multiple_of(wid * CAP + cnt, 8)
        pltpu.sync_copy(ebuf.at[pl.ds(0, KE)],
                        eids_hbm.at[pl.ds(cnt8, KE)])
        pltpu.sync_copy(lbuf.at[pl.ds(0, KE)],
                        ldst_hbm.at[pl.ds(cnt8, KE)])
        cbuf[...] = jnp.broadcast_to(cnt, (16,))
        pltpu.sync_copy(cbuf, cnt_hbm.at[pl.ds(wid * 16, 16)])

    return k(dst)


def _scatter_max(m, s_pad, eids, ldst, cnts):
    h = m.shape[1]
    mesh = plsc.VectorSubcoreMesh(core_axis_name="c", subcore_axis_name="s")

    @functools.partial(
        pl.kernel, mesh=mesh,
        out_type=jax.ShapeDtypeStruct((NP, h), jnp.float32),
        compiler_params=_SC_PARAMS,
        scratch_types=[
            pltpu.VMEM((R + 1, h), jnp.float32),
            pltpu.VMEM((KE, h), jnp.float32),
            pltpu.VMEM((KE,), jnp.int32),
            pltpu.VMEM((KE,), jnp.int32),
            pltpu.VMEM((16,), jnp.int32),
            pltpu.SemaphoreType.DMA,
        ],
    )
    def k(m_hbm, s_hbm, eids_hbm, ldst_hbm, cnt_hbm, out_hbm,
          acc, mrows, eibuf, ldbuf, cbuf, sem):
        wid = lax.axis_index("s") * NC + lax.axis_index("c")
        lo = wid * R
        pltpu.sync_copy(s_hbm.at[pl.ds(lo, R)], acc.at[pl.ds(0, R)])
        pltpu.sync_copy(cnt_hbm.at[pl.ds(wid * 16, 16)], cbuf)
        cnt = cbuf[pl.ds(0, 16)][0]
        nblk = (cnt + KE - 1) // KE

        def chunk(ck, _):
            off = wid * CAP + ck * KE
            pltpu.sync_copy(eids_hbm.at[pl.ds(off, KE)], eibuf)
            pltpu.sync_copy(ldst_hbm.at[pl.ds(off, KE)], ldbuf)
            pltpu.async_copy(m_hbm.at[eibuf], mrows, sem).wait()

            def grp(g, _):
                for b in range(2):
                    ldv = ldbuf[pl.ds(g * 32 + b * 16, 16)]
                    for jj in range(16):
                        d = ldv[jj]
                        r = g * 32 + b * 16 + jj
                        for hk in range(h // 16):
                            sl = pl.ds(hk * 16, 16)
                            acc[d, sl] = jnp.maximum(acc[d, sl],
                                                     mrows[r, sl])
                return ()

            lax.fori_loop(0, KE // 32, grp, (), unroll=False)
            return ()

        lax.fori_loop(0, nblk, chunk, (), unroll=False)
        pltpu.sync_copy(acc.at[pl.ds(0, R)], out_hbm.at[pl.ds(lo, R)])

    return k(m, s_pad, eids, ldst, cnts)


def _layer(f, pos, src, dst, part, Wa, ba, Wb, bb):
    din = f.shape[1]
    waxT = Wa[:, :din].T
    wapT = Wa[:, din:].T
    wbT = Wb.T
    hh = Wa.shape[0]
    hp = max(hh, 128)
    if hp != hh:
        pad = ((0, 0), (0, hp - hh))
        waxT = jnp.pad(waxT, pad)
        wapT = jnp.pad(wapT, pad)
        ba = jnp.pad(ba, (0, hp - hh))
        wbT = jnp.pad(wbT, ((0, hp - hh), (0, hp - hh)))
        bb = jnp.pad(bb, (0, hp - hh))
    u, v, s = _dense_node(f, pos, waxT, wapT, ba.reshape(1, -1), wbT,
                          bb.reshape(1, -1))
    us, vd = _edge_gather(u, v, src, dst)
    m = _edge_mlp(us, vd, wbT, bb.reshape(1, -1))
    s_pad = jnp.pad(s, ((0, NP - s.shape[0]), (0, 0)))
    h_pad = _scatter_max(m, s_pad, *part)
    return h_pad[:N, :hh]


def kernel(x, pos, edge_index, W1a, b1a, W1b, b1b, W2a, b2a, W2b, b2b,
           W3a, b3a, W3b, b3b):
    src = edge_index[0]
    dst = edge_index[1]
    part = _partition(dst)
    h = _layer(x, pos, src, dst, part, W1a, b1a, W1b, b1b)
    h = _layer(h, pos, src, dst, part, W2a, b2a, W2b, b2b)
    h = _layer(h, pos, src, dst, part, W3a, b3a, W3b, b3b)
    return h
